# baseline (speedup 1.0000x reference)
import functools

import jax
import jax.numpy as jnp
from jax import lax
from jax.experimental import pallas as pl
from jax.experimental.pallas import tpu as pltpu
from jax.experimental.pallas import tpu_sc as plsc

NUM_CORES = 2
NUM_SUBCORES = 16
NUM_WORKERS = NUM_CORES * NUM_SUBCORES
CHUNK = 128
EMB = 32
LANES = 16


@jax.jit
def _sc_gather(weight, x_t):
    seq, batch = x_t.shape
    n_stripes = batch // CHUNK
    assert n_stripes == NUM_WORKERS

    mesh = plsc.VectorSubcoreMesh(core_axis_name="c", subcore_axis_name="s")

    @functools.partial(
        pl.kernel,
        mesh=mesh,
        out_type=jax.ShapeDtypeStruct((seq, EMB, batch), jnp.float32),
        scratch_types=[
            pltpu.VMEM((seq, CHUNK), jnp.int32),
            pltpu.VMEM((CHUNK, EMB), jnp.float32),
            pltpu.VMEM((CHUNK, EMB), jnp.float32),
            pltpu.VMEM((EMB, CHUNK), jnp.float32),
            pltpu.VMEM((EMB, CHUNK), jnp.float32),
            pltpu.SemaphoreType.DMA((2,)),
            pltpu.SemaphoreType.DMA((2,)),
        ],
        compiler_params=pltpu.CompilerParams(
            use_tc_tiling_on_sc=False, needs_layout_passes=False
        ),
    )
    def k(table_hbm, xt_hbm, out_hbm, idx_v, rb0, rb1, tb0, tb1, gsem, psem):
        wid = lax.axis_index("s") * NUM_CORES + lax.axis_index("c")
        col0 = wid * CHUNK

        pltpu.sync_copy(xt_hbm.at[pl.ds(0, seq), pl.ds(col0, CHUNK)], idx_v)

        rbufs = (rb0, rb1)
        tbufs = (tb0, tb1)

        def fire_gather(s, b):
            pltpu.async_copy(
                table_hbm.at[idx_v.at[s]], rbufs[b], gsem.at[b]
            )

        def wait_gather(b):
            pltpu.make_async_copy(
                table_hbm.at[pl.ds(0, CHUNK)], rbufs[b], gsem.at[b]
            ).wait()

        def out_slice(s):
            return out_hbm.at[s].at[pl.ds(0, EMB), pl.ds(col0, CHUNK)]

        def fire_put(s, b):
            pltpu.async_copy(tbufs[b], out_slice(s), psem.at[b])

        def wait_put(b):
            pltpu.make_async_copy(tbufs[b], out_slice(0), psem.at[b]).wait()

        def transpose(b):
            rb, tb = rbufs[b], tbufs[b]
            for f in range(EMB):
                fv = jnp.full((LANES,), f, jnp.int32)
                for q in range(CHUNK // LANES):
                    jv = lax.iota(jnp.int32, LANES) + (q * LANES)
                    tb[f, pl.ds(q * LANES, LANES)] = plsc.load_gather(
                        rb, [jv, fv]
                    )

        fire_gather(0, 0)
        wait_gather(0)
        fire_gather(1, 1)
        transpose(0)
        fire_put(0, 0)
        wait_gather(1)
        fire_gather(2, 0)
        transpose(1)
        fire_put(1, 1)

        def body(i, carry):
            s = 2 * i
            wait_gather(0)
            fire_gather(s + 1, 1)
            wait_put(0)
            transpose(0)
            fire_put(s, 0)
            wait_gather(1)
            fire_gather(s + 2, 0)
            wait_put(1)
            transpose(1)
            fire_put(s + 1, 1)
            return carry

        lax.fori_loop(1, seq // 2 - 1, body, 0)

        wait_gather(0)
        fire_gather(seq - 1, 1)
        wait_put(0)
        transpose(0)
        fire_put(seq - 2, 0)
        wait_gather(1)
        wait_put(1)
        transpose(1)
        fire_put(seq - 1, 1)
        wait_put(0)
        wait_put(1)

    return k(weight, x_t)


def kernel(x, weight):
    b, s = x.shape
    out_phys = _sc_gather(weight, x.T)
    return out_phys.transpose(2, 0, 1)


# speedup vs baseline: 1.2406x; 1.2406x over previous
import functools

import jax
import jax.numpy as jnp
from jax import lax
from jax.experimental import pallas as pl
from jax.experimental.pallas import tpu as pltpu
from jax.experimental.pallas import tpu_sc as plsc

NUM_CORES = 2
NUM_SUBCORES = 16
NUM_WORKERS = NUM_CORES * NUM_SUBCORES
CHUNK = 128
EMB = 32
LANES = 16


@jax.jit
def _sc_gather(weight, x_t):
    seq, batch = x_t.shape
    n_stripes = batch // CHUNK
    assert n_stripes == NUM_WORKERS

    mesh = plsc.VectorSubcoreMesh(core_axis_name="c", subcore_axis_name="s")

    @functools.partial(
        pl.kernel,
        mesh=mesh,
        out_type=jax.ShapeDtypeStruct((seq, EMB, batch), jnp.float32),
        scratch_types=[
            pltpu.VMEM((seq, CHUNK), jnp.int32),
            pltpu.VMEM((CHUNK, EMB), jnp.float32),
            pltpu.VMEM((CHUNK, EMB), jnp.float32),
            pltpu.VMEM((EMB, CHUNK), jnp.float32),
            pltpu.VMEM((EMB, CHUNK), jnp.float32),
            pltpu.SemaphoreType.DMA((2,)),
            pltpu.SemaphoreType.DMA((2,)),
        ],
        compiler_params=pltpu.CompilerParams(
            use_tc_tiling_on_sc=False, needs_layout_passes=False
        ),
    )
    def k(table_hbm, xt_hbm, out_hbm, idx_v, rb0, rb1, tb0, tb1, gsem, psem):
        wid = lax.axis_index("s") * NUM_CORES + lax.axis_index("c")
        col0 = wid * CHUNK

        pltpu.sync_copy(xt_hbm.at[pl.ds(0, seq), pl.ds(col0, CHUNK)], idx_v)

        rbufs = (rb0, rb1)
        tbufs = (tb0, tb1)

        def fire_gather(s, b):
            pltpu.async_copy(
                table_hbm.at[idx_v.at[s]], rbufs[b], gsem.at[b]
            )

        def wait_gather(b):
            pltpu.make_async_copy(
                table_hbm.at[pl.ds(0, CHUNK)], rbufs[b], gsem.at[b]
            ).wait()

        def out_slice(s):
            return out_hbm.at[s].at[pl.ds(0, EMB), pl.ds(col0, CHUNK)]

        def fire_put(s, b):
            pltpu.async_copy(tbufs[b], out_slice(s), psem.at[b])

        def wait_put(b):
            pltpu.make_async_copy(tbufs[b], out_slice(0), psem.at[b]).wait()

        nq = CHUNK // LANES
        base_iota = lax.iota(jnp.int32, LANES)
        DEPTH = 8

        def transpose(b):
            rb, tb = rbufs[b], tbufs[b]
            jv = base_iota
            for q in range(nq):
                if q:
                    jv = jv + LANES
                fv = jnp.zeros((LANES,), jnp.int32)
                pend = {}
                for f in range(EMB + DEPTH):
                    if f < EMB:
                        pend[f] = plsc.load_gather(rb, [jv, fv])
                        fv = fv + 1
                    if f >= DEPTH:
                        tb[f - DEPTH, pl.ds(q * LANES, LANES)] = pend.pop(
                            f - DEPTH
                        )

        fire_gather(0, 0)
        wait_gather(0)
        fire_gather(1, 1)
        transpose(0)
        fire_put(0, 0)
        wait_gather(1)
        fire_gather(2, 0)
        transpose(1)
        fire_put(1, 1)

        def body(i, carry):
            s = 2 * i
            wait_gather(0)
            fire_gather(s + 1, 1)
            wait_put(0)
            transpose(0)
            fire_put(s, 0)
            wait_gather(1)
            fire_gather(s + 2, 0)
            wait_put(1)
            transpose(1)
            fire_put(s + 1, 1)
            return carry

        lax.fori_loop(1, seq // 2 - 1, body, 0)

        wait_gather(0)
        fire_gather(seq - 1, 1)
        wait_put(0)
        transpose(0)
        fire_put(seq - 2, 0)
        wait_gather(1)
        wait_put(1)
        transpose(1)
        fire_put(seq - 1, 1)
        wait_put(0)
        wait_put(1)

    return k(weight, x_t)


def kernel(x, weight):
    b, s = x.shape
    out_phys = _sc_gather(weight, x.T)
    return out_phys.transpose(2, 0, 1)


# speedup vs baseline: 1.6086x; 1.2966x over previous
import functools

import jax
import jax.numpy as jnp
from jax import lax
from jax.experimental import pallas as pl
from jax.experimental.pallas import tpu as pltpu
from jax.experimental.pallas import tpu_sc as plsc

NUM_CORES = 2
NUM_SUBCORES = 16
NUM_WORKERS = NUM_CORES * NUM_SUBCORES
CHUNK = 128
EMB = 32
LANES = 16


@jax.jit
def _sc_gather(weight, x_t):
    seq, batch = x_t.shape
    assert batch // CHUNK == NUM_WORKERS

    mesh = plsc.VectorSubcoreMesh(core_axis_name="c", subcore_axis_name="s")

    @functools.partial(
        pl.kernel,
        mesh=mesh,
        out_type=jax.ShapeDtypeStruct((seq, EMB, batch), jnp.float32),
        scratch_types=[
            pltpu.VMEM((seq, CHUNK), jnp.int32),
            pltpu.VMEM((2, CHUNK, EMB), jnp.float32),
            pltpu.VMEM((2, EMB, CHUNK), jnp.float32),
            pltpu.SemaphoreType.DMA((2,)),
            pltpu.SemaphoreType.DMA((2,)),
        ],
        compiler_params=pltpu.CompilerParams(
            use_tc_tiling_on_sc=False, needs_layout_passes=False
        ),
    )
    def k(table_hbm, xt_hbm, out_hbm, idx_v, rows_v, tps_v, gsem, psem):
        wid = lax.axis_index("s") * NUM_CORES + lax.axis_index("c")
        col0 = wid * CHUNK

        pltpu.sync_copy(xt_hbm.at[pl.ds(0, seq), pl.ds(col0, CHUNK)], idx_v)

        def fire_gather(s, b):
            pltpu.async_copy(
                table_hbm.at[idx_v.at[s]], rows_v.at[b], gsem.at[b]
            )

        def wait_gather(b):
            pltpu.make_async_copy(
                table_hbm.at[pl.ds(0, CHUNK)], rows_v.at[b], gsem.at[b]
            ).wait()

        def out_slice(s):
            return out_hbm.at[s].at[pl.ds(0, EMB), pl.ds(col0, CHUNK)]

        def fire_put(s, b):
            pltpu.async_copy(tps_v.at[b], out_slice(s), psem.at[b])

        def wait_put(b):
            pltpu.make_async_copy(
                tps_v.at[b], out_slice(0), psem.at[b]
            ).wait()

        base_iota = lax.iota(jnp.int32, LANES)
        DEPTH = 4

        def transpose(b):
            bv = jnp.full((LANES,), 0, jnp.int32) + b
            for jb in range(CHUNK // LANES):
                jv = base_iota + (jb * LANES) if jb else base_iota
                for fh in range(EMB // LANES):
                    f0 = fh * LANES
                    rot = base_iota
                    pend = []
                    for d in range(LANES + DEPTH):
                        if d < LANES:
                            fv = rot + f0 if f0 else rot
                            pend.append(
                                (fv, plsc.load_gather(rows_v, [bv, jv, fv]))
                            )
                            rot = (rot + 1) & (LANES - 1)
                        if d >= DEPTH:
                            fv_o, v_o = pend[d - DEPTH]
                            plsc.store_scatter(tps_v, [bv, fv_o, jv], v_o)

        fire_gather(0, 0)
        fire_put(0, 0)
        fire_put(1, 1)

        def body(s, carry):
            b = lax.rem(s, 2)
            b2 = 1 - b
            wait_gather(b)
            fire_gather(lax.min(s + 1, seq - 1), b2)
            wait_put(b)
            transpose(b)
            fire_put(s, b)
            return carry

        lax.fori_loop(0, seq, body, 0)

        wait_gather(seq % 2)
        wait_put(0)
        wait_put(1)

    return k(weight, x_t)


def kernel(x, weight):
    out_phys = _sc_gather(weight, x.T)
    return out_phys.transpose(2, 0, 1)
